# Initial kernel scaffold; baseline (speedup 1.0000x reference)
import jax
import jax.numpy as jnp
from jax.experimental import pallas as pl


def kernel(x, edge_index, emb_W, emb_b, Wq, bq, Wk, bk, Wv, bv, Ws, bs, ln_g, ln_b, out_W, out_b):
    raise NotImplementedError("write your pallas kernel here")


# speedup vs baseline: 7.4065x; 7.4065x over previous
import functools
import math

import jax
import jax.numpy as jnp
from jax import lax
from jax.experimental import pallas as pl
from jax.experimental.pallas import tpu as pltpu
from jax.experimental.pallas import tpu_sc as plsc

N = 10000
E = 160000
D = 128
H = 4
C = 128
HC = H * C
NP = 10240
NC = 2
NS = 16
NW = NC * NS
EPC = E // NC
EPW = E // NW
BS = 200
ROWS_PER_SUB = NP // NS


def _mm_bias(x, W, b, bn=1024):
    M, K = x.shape
    F = W.shape[1]

    def body(x_ref, w_ref, b_ref, o_ref):
        o_ref[...] = (
            jnp.dot(x_ref[...], w_ref[...], preferred_element_type=jnp.float32)
            + b_ref[...]
        )

    return pl.pallas_call(
        body,
        grid=(M // bn,),
        in_specs=[
            pl.BlockSpec((bn, K), lambda i: (i, 0)),
            pl.BlockSpec((K, F), lambda i: (0, 0)),
            pl.BlockSpec((1, F), lambda i: (0, 0)),
        ],
        out_specs=pl.BlockSpec((bn, F), lambda i: (i, 0)),
        out_shape=jax.ShapeDtypeStruct((M, F), jnp.float32),
    )(x, W, b.reshape(1, F))


def _qkvs_mm(h, W_cat, b_cat, bn=1024):
    F = HC * 3 + D

    def body(h_ref, w_ref, b_ref, q_ref, k_ref, v_ref, s_ref):
        res = (
            jnp.dot(h_ref[...], w_ref[...], preferred_element_type=jnp.float32)
            + b_ref[...]
        )
        q_ref[...] = res[:, :HC]
        k_ref[...] = res[:, HC:2 * HC]
        v_ref[...] = res[:, 2 * HC:3 * HC]
        s_ref[...] = res[:, 3 * HC:]

    return pl.pallas_call(
        body,
        grid=(NP // bn,),
        in_specs=[
            pl.BlockSpec((bn, D), lambda i: (i, 0)),
            pl.BlockSpec((D, F), lambda i: (0, 0)),
            pl.BlockSpec((1, F), lambda i: (0, 0)),
        ],
        out_specs=[
            pl.BlockSpec((bn, HC), lambda i: (i, 0)),
            pl.BlockSpec((bn, HC), lambda i: (i, 0)),
            pl.BlockSpec((bn, HC), lambda i: (i, 0)),
            pl.BlockSpec((bn, D), lambda i: (i, 0)),
        ],
        out_shape=[
            jax.ShapeDtypeStruct((NP, HC), jnp.float32),
            jax.ShapeDtypeStruct((NP, HC), jnp.float32),
            jax.ShapeDtypeStruct((NP, HC), jnp.float32),
            jax.ShapeDtypeStruct((NP, D), jnp.float32),
        ],
    )(h, W_cat, b_cat.reshape(1, F))


def _alpha_weight(gq, gk, gv, be=256):
    inv = 1.0 / math.sqrt(float(C))

    def body(q_ref, k_ref, v_ref, wv_ref):
        cols = []
        for h in range(H):
            sl = slice(h * C, (h + 1) * C)
            a = jnp.sum(q_ref[:, sl] * k_ref[:, sl], axis=1, keepdims=True)
            w = jnp.exp(a * inv)
            wv_ref[h, :, :] = v_ref[:, sl] * w
            cols.append(w)
        cols.append(jnp.zeros((be, C - H), jnp.float32))
        wv_ref[H, :, :] = jnp.concatenate(cols, axis=1)

    return pl.pallas_call(
        body,
        grid=(E // be,),
        in_specs=[
            pl.BlockSpec((be, HC), lambda i: (i, 0)),
            pl.BlockSpec((be, HC), lambda i: (i, 0)),
            pl.BlockSpec((be, HC), lambda i: (i, 0)),
        ],
        out_specs=pl.BlockSpec((H + 1, be, C), lambda i: (0, i, 0)),
        out_shape=jax.ShapeDtypeStruct((H + 1, E, C), jnp.float32),
    )(gq, gk, gv)


def _finish(num_part, hs, bs, g, b, bn=512):
    def body(p_ref, s_ref, bs_ref, g_ref, b_ref, o_ref):
        den = p_ref[0, H] + p_ref[1, H]
        acc = jnp.zeros((bn, C), jnp.float32)
        for h in range(H):
            num = p_ref[0, h] + p_ref[1, h]
            dh = den[:, h:h + 1] + 1e-16
            acc = acc + num / dh
        t = acc * (1.0 / H) + s_ref[...] + bs_ref[...]
        t = jnp.maximum(t, 0.0)
        mu = jnp.mean(t, axis=1, keepdims=True)
        var = jnp.mean((t - mu) ** 2, axis=1, keepdims=True)
        o_ref[...] = (t - mu) / jnp.sqrt(var + 1e-5) * g_ref[...] + b_ref[...]

    return pl.pallas_call(
        body,
        grid=(NP // bn,),
        in_specs=[
            pl.BlockSpec((2, H + 1, bn, C), lambda i: (0, 0, i, 0)),
            pl.BlockSpec((bn, D), lambda i: (i, 0)),
            pl.BlockSpec((1, D), lambda i: (0, 0)),
            pl.BlockSpec((1, D), lambda i: (0, 0)),
            pl.BlockSpec((1, D), lambda i: (0, 0)),
        ],
        out_specs=pl.BlockSpec((bn, D), lambda i: (i, 0)),
        out_shape=jax.ShapeDtypeStruct((NP, D), jnp.float32),
    )(num_part, hs, bs.reshape(1, D), g.reshape(1, D), b.reshape(1, D))


def _head(h, out_W, out_b, bn=1024):
    def body(h_ref, w_ref, b_ref, o_ref):
        t = jnp.sum(h_ref[...] * w_ref[...], axis=1, keepdims=True)
        o_ref[...] = jax.nn.sigmoid(t + b_ref[0, 0])

    return pl.pallas_call(
        body,
        grid=(NP // bn,),
        in_specs=[
            pl.BlockSpec((bn, D), lambda i: (i, 0)),
            pl.BlockSpec((1, D), lambda i: (0, 0)),
            pl.BlockSpec((1, 1), lambda i: (0, 0)),
        ],
        out_specs=pl.BlockSpec((bn, 1), lambda i: (i, 0)),
        out_shape=jax.ShapeDtypeStruct((NP, 1), jnp.float32),
    )(h, out_W.reshape(1, D), out_b.reshape(1, 1))


_MESH = plsc.VectorSubcoreMesh(core_axis_name="c", subcore_axis_name="s")


def _sc_gather(q, k, v, dst, src):
    @functools.partial(
        pl.kernel,
        out_type=(
            jax.ShapeDtypeStruct((E, HC), jnp.float32),
            jax.ShapeDtypeStruct((E, HC), jnp.float32),
            jax.ShapeDtypeStruct((E, HC), jnp.float32),
        ),
        mesh=_MESH,
        scratch_types=[
            pltpu.VMEM((BS,), jnp.int32),
            pltpu.VMEM((BS,), jnp.int32),
            pltpu.VMEM((BS, HC), jnp.float32),
            pltpu.SemaphoreType.DMA,
        ],
    )
    def kern(q_hbm, k_hbm, v_hbm, dst_hbm, src_hbm, gq_hbm, gk_hbm, gv_hbm,
             didx_v, sidx_v, rows_v, sem):
        c = lax.axis_index("c")
        s = lax.axis_index("s")
        ebase = c * EPC + s * EPW

        @pl.loop(0, EPW // BS)
        def _g(j):
            base = ebase + j * BS
            pltpu.sync_copy(dst_hbm.at[pl.ds(base, BS)], didx_v)
            pltpu.sync_copy(src_hbm.at[pl.ds(base, BS)], sidx_v)
            pltpu.async_copy(q_hbm.at[didx_v], rows_v, sem).wait()
            pltpu.sync_copy(rows_v, gq_hbm.at[pl.ds(base, BS)])
            pltpu.async_copy(k_hbm.at[sidx_v], rows_v, sem).wait()
            pltpu.sync_copy(rows_v, gk_hbm.at[pl.ds(base, BS)])
            pltpu.async_copy(v_hbm.at[sidx_v], rows_v, sem).wait()
            pltpu.sync_copy(rows_v, gv_hbm.at[pl.ds(base, BS)])

    return kern(q, k, v, dst, src)


def _sc_scatter(wv, dst, zeros_c):
    @functools.partial(
        pl.kernel,
        out_type=jax.ShapeDtypeStruct((NC, H + 1, NP, C), jnp.float32),
        mesh=_MESH,
        scratch_types=[
            pltpu.VMEM((BS,), jnp.int32),
            pltpu.VMEM((BS, C), jnp.float32),
            pltpu.VMEM_SHARED((NP, C), jnp.float32),
        ],
    )
    def kern(wv_hbm, dst_hbm, z_hbm, num_out, idx_v, rows_v, acc_s):
        c = lax.axis_index("c")
        s = lax.axis_index("s")
        row0 = s * ROWS_PER_SUB
        ebase = c * EPC + s * EPW

        for h in range(H + 1):
            pltpu.sync_copy(z_hbm.at[pl.ds(row0, ROWS_PER_SUB)],
                            acc_s.at[pl.ds(row0, ROWS_PER_SUB)])
            plsc.subcore_barrier()

            @pl.loop(0, EPW // BS)
            def _num(j):
                base = ebase + j * BS
                pltpu.sync_copy(dst_hbm.at[pl.ds(base, BS)], idx_v)
                pltpu.sync_copy(wv_hbm.at[h, pl.ds(base, BS)], rows_v)
                pltpu.sync_copy(rows_v, acc_s.at[idx_v], add=True)

            plsc.subcore_barrier()
            pltpu.sync_copy(acc_s.at[pl.ds(row0, ROWS_PER_SUB)],
                            num_out.at[c, h, pl.ds(row0, ROWS_PER_SUB)])
            plsc.subcore_barrier()

    return kern(wv, dst, zeros_c)


def kernel(x, edge_index, emb_W, emb_b, Wq, bq, Wk, bk, Wv, bv, Ws, bs,
           ln_g, ln_b, out_W, out_b):
    src = edge_index[0]
    dst = edge_index[1]

    x_pad = jnp.pad(x, ((0, NP - N), (0, 0)))
    zeros_c = jnp.zeros((NP, C), jnp.float32)

    h = _mm_bias(x_pad, emb_W, emb_b)

    for l in range(L_LAYERS):
        W_cat = jnp.concatenate([Wq[l], Wk[l], Wv[l], Ws[l]], axis=1)
        b_cat = jnp.concatenate([bq[l], bk[l], bv[l], bs[l]], axis=0)
        q, k, v, hs = _qkvs_mm(h, W_cat, b_cat)
        gq, gk, gv = _sc_gather(q, k, v, dst, src)
        wv = _alpha_weight(gq, gk, gv)
        num_part = _sc_scatter(wv, dst, zeros_c)
        h = _finish(num_part, hs, bs[l], ln_g[l], ln_b[l])

    o = _head(h, out_W, out_b)
    return o[:N, 0]


L_LAYERS = 4
